# baseline (speedup 1.0000x reference)
import jax
import jax.numpy as jnp
from jax import lax
from jax.experimental import pallas as pl
from jax.experimental.pallas import tpu as pltpu
from jax.experimental.pallas import tpu_sc as plsc

NC = 2
NS = 16
NW = NC * NS
L = 16

B, T, D = 16, 2048, 512
ROWS_W = T // NW


def _sc_trim(seq_hbm, pe_hbm, out_hbm, pe_v, buf0, buf1, pe_sem, in_sems, out_sems):
    wid = lax.axis_index("s") * NC + lax.axis_index("c")
    row0 = wid * ROWS_W
    bufs = (buf0, buf1)

    pltpu.async_copy(pe_hbm.at[0, pl.ds(row0, ROWS_W)], pe_v, pe_sem).wait()

    def in_dma(b, slot):
        return pltpu.make_async_copy(
            seq_hbm.at[b, pl.ds(row0, ROWS_W)], bufs[slot], in_sems.at[slot]
        )

    def out_dma(b, slot):
        return pltpu.make_async_copy(
            bufs[slot], out_hbm.at[b, 0, pl.ds(row0, ROWS_W)], out_sems.at[slot]
        )

    in_dma(0, 0).start()
    for b in range(B):
        slot = b % 2
        nxt = (b + 1) % 2
        if b + 1 < B:
            if b >= 1:
                out_dma(b - 1, nxt).wait()
            in_dma(b + 1, nxt).start()
        in_dma(b, slot).wait()

        buf = bufs[slot]

        @pl.loop(0, ROWS_W)
        def _row(r):
            @plsc.parallel_loop(0, D, step=L, unroll=8)
            def _col(c):
                buf[r, pl.ds(c, L)] = buf[r, pl.ds(c, L)] * 2.0 + pe_v[r, pl.ds(c, L)]

        out_dma(b, slot).start()

    out_dma(B - 2, 0 if (B - 2) % 2 == 0 else 1).wait()
    out_dma(B - 1, (B - 1) % 2).wait()


def kernel(seq, times, pe):
    del times
    mesh = plsc.VectorSubcoreMesh(core_axis_name="c", subcore_axis_name="s")
    out = pl.kernel(
        _sc_trim,
        out_type=jax.ShapeDtypeStruct((B, 1, T, D), jnp.float32),
        mesh=mesh,
        scratch_types=[
            pltpu.VMEM((ROWS_W, D), jnp.float32),
            pltpu.VMEM((ROWS_W, D), jnp.float32),
            pltpu.VMEM((ROWS_W, D), jnp.float32),
            pltpu.SemaphoreType.DMA,
            pltpu.SemaphoreType.DMA((2,)),
            pltpu.SemaphoreType.DMA((2,)),
        ],
    )(seq, pe)
    mask = jnp.ones((B, 1), dtype=bool)
    return (out, mask)
